# Initial kernel scaffold; baseline (speedup 1.0000x reference)
import jax
import jax.numpy as jnp
from jax.experimental import pallas as pl


def kernel(x, train_x, train_y):
    raise NotImplementedError("write your pallas kernel here")


# speedup vs baseline: 5.1080x; 5.1080x over previous
import functools

import jax
import jax.numpy as jnp
from jax import lax
from jax.experimental import pallas as pl
from jax.experimental.pallas import tpu as pltpu
from jax.experimental.pallas import tpu_sc as plsc

Q = 1024
D = 128
N = 100000
CH = 128
TB = 2048
QB = 256
NPAD = 100352
NJ = NPAD // TB
NCH = NPAD // CH
NCHP = 896
KCH = 16
CAND = KCH * CH
NN = 15
NCLS = 10

_BIG_F = 3.0e38
_BIG_I = 1 << 30


def _dist_body(x_ref, tx_ref, xsq_ref, tsq_ref, d2_ref, cm_ref):
    xb = x_ref[...]
    tb = tx_ref[...]
    mm = lax.dot_general(xb, tb, (((1,), (1,)), ((), ())),
                         preferred_element_type=jnp.float32)
    tsq = tsq_ref[0, 0, :][None, :]
    xsq = xsq_ref[:, 0:1]
    d2 = (xsq + tsq) - 2.0 * mm
    d2_ref[...] = d2
    lane = lax.broadcasted_iota(jnp.int32, (QB, 128), 1)
    cm = jnp.zeros((QB, 128), jnp.float32)
    for c in range(TB // CH):
        mins = jnp.min(d2[:, c * CH:(c + 1) * CH], axis=1)
        cm = cm + jnp.where(lane == c, mins[:, None], 0.0)
    cm_ref[...] = cm.reshape(1, QB, 128)


def _distances(x, tx_pad, xsq_t, tsq_r):
    return pl.pallas_call(
        _dist_body,
        grid=(NJ, Q // QB),
        in_specs=[
            pl.BlockSpec((QB, D), lambda j, q: (q, 0)),
            pl.BlockSpec((TB, D), lambda j, q: (j, 0)),
            pl.BlockSpec((QB, 128), lambda j, q: (q, 0)),
            pl.BlockSpec((1, 1, TB), lambda j, q: (j, 0, 0)),
        ],
        out_specs=[
            pl.BlockSpec((QB, TB), lambda j, q: (q, j)),
            pl.BlockSpec((1, QB, 128), lambda j, q: (j, q, 0)),
        ],
        out_shape=[
            jax.ShapeDtypeStruct((Q, NPAD), jnp.float32),
            jax.ShapeDtypeStruct((NJ, Q, 128), jnp.float32),
        ],
    )(x, tx_pad, xsq_t, tsq_r)


def _select_body(cm_ref, ids_ref, fidx_ref):
    w = cm_ref[...]
    col = lax.broadcasted_iota(jnp.int32, (QB, NCHP), 1)
    lane = lax.broadcasted_iota(jnp.int32, (QB, 128), 1)
    ids = jnp.zeros((QB, 128), jnp.int32)
    for i in range(KCH):
        m = jnp.min(w, axis=1, keepdims=True)
        first = jnp.min(jnp.where(w == m, col, _BIG_I), axis=1, keepdims=True)
        ids = ids + jnp.where(lane == i, first, 0)
        w = jnp.where(col == first, _BIG_F, w)
    s = jnp.where(lane < KCH, ids, _BIG_I)
    srt = jnp.zeros((QB, 128), jnp.int32)
    for j in range(KCH):
        mn = jnp.min(s, axis=1, keepdims=True)
        srt = srt + jnp.where(lane == j, mn, 0)
        s = jnp.where(s == mn, _BIG_I, s)
    qrow = (lax.broadcasted_iota(jnp.int32, (QB, 128), 0)
            + pl.program_id(0) * QB)
    valid = lane < KCH
    ids_ref[...] = jnp.where(valid, srt, 0)
    fidx_ref[...] = jnp.where(valid, srt + qrow * NCH, 0)


def _select_chunks(cm2d):
    return pl.pallas_call(
        _select_body,
        grid=(Q // QB,),
        in_specs=[pl.BlockSpec((QB, NCHP), lambda q: (q, 0))],
        out_specs=[
            pl.BlockSpec((QB, 128), lambda q: (q, 0)),
            pl.BlockSpec((QB, 128), lambda q: (q, 0)),
        ],
        out_shape=[
            jax.ShapeDtypeStruct((Q, 128), jnp.int32),
            jax.ShapeDtypeStruct((Q, 128), jnp.int32),
        ],
    )(cm2d)


_NC = 2
_NS = 16
_NW = _NC * _NS
_ROWS = Q * KCH
_RPW = _ROWS // _NW
_SUB = 128


def _gather_body(d2_tab, y_tab, idx_d2, idx_lab,
                 out_d2, out_lab, idx_v, rows_f, rows_i, sem):
    wid = lax.axis_index("s") * _NC + lax.axis_index("c")
    for b in range(_RPW // _SUB):
        base = wid * _RPW + b * _SUB
        pltpu.sync_copy(idx_d2.at[pl.ds(base, _SUB)], idx_v)
        pltpu.async_copy(d2_tab.at[idx_v], rows_f, sem).wait()
        pltpu.sync_copy(rows_f, out_d2.at[pl.ds(base, _SUB)])
        pltpu.sync_copy(idx_lab.at[pl.ds(base, _SUB)], idx_v)
        pltpu.async_copy(y_tab.at[idx_v], rows_i, sem).wait()
        pltpu.sync_copy(rows_i, out_lab.at[pl.ds(base, _SUB)])


def _gather_candidates(d2_tab, y_tab, idx_d2, idx_lab):
    mesh = plsc.VectorSubcoreMesh(core_axis_name="c", subcore_axis_name="s")
    f = functools.partial(
        pl.kernel,
        mesh=mesh,
        out_type=[
            jax.ShapeDtypeStruct((_ROWS, CH), jnp.float32),
            jax.ShapeDtypeStruct((_ROWS, CH), jnp.int32),
        ],
        scratch_types=[
            pltpu.VMEM((_SUB,), jnp.int32),
            pltpu.VMEM((_SUB, CH), jnp.float32),
            pltpu.VMEM((_SUB, CH), jnp.int32),
            pltpu.SemaphoreType.DMA,
        ],
    )(_gather_body)
    return f(d2_tab, y_tab, idx_d2, idx_lab)


def _vote_body(d2c_ref, lab_ref, preds_ref, probs_ref):
    v = d2c_ref[...]
    labs = lab_ref[...]
    lane = lax.broadcasted_iota(jnp.int32, (QB, CAND), 1)
    cls = lax.broadcasted_iota(jnp.int32, (QB, 128), 1)
    votes = jnp.zeros((QB, 128), jnp.float32)
    for _ in range(NN):
        m = jnp.min(v, axis=1, keepdims=True)
        pos = jnp.min(jnp.where(v == m, lane, _BIG_I), axis=1, keepdims=True)
        sel = lane == pos
        labsel = jnp.sum(jnp.where(sel, labs, 0), axis=1, keepdims=True)
        votes = votes + jnp.where(cls == labsel, 1.0, 0.0)
        v = jnp.where(sel, _BIG_F, v)
    probs_ref[...] = votes / float(NN)
    pv = jnp.where(cls < NCLS, votes, -1.0)
    mx = jnp.max(pv, axis=1, keepdims=True)
    pred = jnp.min(jnp.where(pv == mx, cls, _BIG_I), axis=1, keepdims=True)
    preds_ref[...] = jnp.broadcast_to(pred, (QB, 128))


def _vote(cand_d2, cand_lab):
    return pl.pallas_call(
        _vote_body,
        grid=(Q // QB,),
        in_specs=[
            pl.BlockSpec((QB, CAND), lambda q: (q, 0)),
            pl.BlockSpec((QB, CAND), lambda q: (q, 0)),
        ],
        out_specs=[
            pl.BlockSpec((QB, 128), lambda q: (q, 0)),
            pl.BlockSpec((QB, 128), lambda q: (q, 0)),
        ],
        out_shape=[
            jax.ShapeDtypeStruct((Q, 128), jnp.int32),
            jax.ShapeDtypeStruct((Q, 128), jnp.float32),
        ],
    )(cand_d2, cand_lab)


def kernel(x, train_x, train_y):
    x_sq = jnp.sum(x * x, axis=1, keepdims=True)
    t_sq = jnp.sum(train_x * train_x, axis=1)
    xsq_t = jnp.broadcast_to(x_sq, (Q, 128))
    t_sq_pad = jnp.concatenate(
        [t_sq, jnp.full((NPAD - N,), 1e9, jnp.float32)]).reshape(NJ, 1, TB)
    tx_pad = jnp.concatenate(
        [train_x, jnp.zeros((NPAD - N, D), jnp.float32)], axis=0)
    ty_pad = jnp.concatenate(
        [train_y, jnp.zeros((NPAD - N,), jnp.int32)]).reshape(NCH, CH)

    d2, cm3 = _distances(x, tx_pad, xsq_t, t_sq_pad)
    cm2d = cm3.transpose(1, 0, 2)[:, :, :TB // CH].reshape(Q, NCH)
    cm2d = jnp.concatenate(
        [cm2d, jnp.full((Q, NCHP - NCH), 2e30, jnp.float32)], axis=1)

    ids_pad, fidx_pad = _select_chunks(cm2d)
    idx_d2 = fidx_pad[:, :KCH].reshape(_ROWS)
    idx_lab = ids_pad[:, :KCH].reshape(_ROWS)

    cand_d2, cand_lab = _gather_candidates(
        d2.reshape(Q * NCH, CH), ty_pad, idx_d2, idx_lab)

    preds_pad, probs_pad = _vote(
        cand_d2.reshape(Q, CAND), cand_lab.reshape(Q, CAND))
    return preds_pad[:, 0], probs_pad[:, :NCLS]


# speedup vs baseline: 5.3147x; 1.0405x over previous
import functools

import jax
import jax.numpy as jnp
from jax import lax
from jax.experimental import pallas as pl
from jax.experimental.pallas import tpu as pltpu
from jax.experimental.pallas import tpu_sc as plsc

Q = 1024
D = 128
N = 100000
CH = 128
TB = 2048
QB = 256
NPAD = 100352
NJ = NPAD // TB
NCH = NPAD // CH
NCHP = 896
KCH = 16
CAND = KCH * CH
NN = 15
NCLS = 10

_BIG_F = 3.0e38
_BIG_I = 1 << 30


def _dist_body(x_ref, tx_ref, xsq_ref, tsq_ref, d2_ref, cm_ref):
    xb = x_ref[...]
    tb = tx_ref[...]
    mm = lax.dot_general(xb, tb, (((1,), (1,)), ((), ())),
                         preferred_element_type=jnp.float32)
    tsq = tsq_ref[0, 0, :][None, :]
    xsq = xsq_ref[:, 0:1]
    d2 = (xsq + tsq) - 2.0 * mm
    gcol = lax.broadcasted_iota(jnp.int32, (QB, TB), 1) + pl.program_id(0) * TB
    d2 = jnp.where(gcol < N, d2, 1e9)
    d2_ref[...] = d2
    lane = lax.broadcasted_iota(jnp.int32, (QB, 128), 1)
    cm = jnp.zeros((QB, 128), jnp.float32)
    for c in range(TB // CH):
        mins = jnp.min(d2[:, c * CH:(c + 1) * CH], axis=1)
        cm = cm + jnp.where(lane == c, mins[:, None], 0.0)
    cm_ref[...] = cm.reshape(1, QB, 128)


def _distances(x, tx_pad, xsq_t, tsq_r):
    return pl.pallas_call(
        _dist_body,
        grid=(NJ, Q // QB),
        in_specs=[
            pl.BlockSpec((QB, D), lambda j, q: (q, 0)),
            pl.BlockSpec((TB, D), lambda j, q: (j, 0)),
            pl.BlockSpec((QB, 128), lambda j, q: (q, 0)),
            pl.BlockSpec((1, 1, TB), lambda j, q: (j, 0, 0)),
        ],
        out_specs=[
            pl.BlockSpec((QB, TB), lambda j, q: (q, j)),
            pl.BlockSpec((1, QB, 128), lambda j, q: (j, q, 0)),
        ],
        out_shape=[
            jax.ShapeDtypeStruct((Q, NPAD), jnp.float32),
            jax.ShapeDtypeStruct((NJ, Q, 128), jnp.float32),
        ],
    )(x, tx_pad, xsq_t, tsq_r)


def _select_body(cm_ref, ids_ref, fidx_ref):
    w = cm_ref[...]
    col = lax.broadcasted_iota(jnp.int32, (QB, NCHP), 1)
    lane = lax.broadcasted_iota(jnp.int32, (QB, 128), 1)
    ids = jnp.zeros((QB, 128), jnp.int32)
    for i in range(KCH):
        m = jnp.min(w, axis=1, keepdims=True)
        first = jnp.min(jnp.where(w == m, col, _BIG_I), axis=1, keepdims=True)
        ids = ids + jnp.where(lane == i, first, 0)
        w = jnp.where(col == first, _BIG_F, w)
    s = jnp.where(lane < KCH, ids, _BIG_I)
    srt = jnp.zeros((QB, 128), jnp.int32)
    for j in range(KCH):
        mn = jnp.min(s, axis=1, keepdims=True)
        srt = srt + jnp.where(lane == j, mn, 0)
        s = jnp.where(s == mn, _BIG_I, s)
    qrow = (lax.broadcasted_iota(jnp.int32, (QB, 128), 0)
            + pl.program_id(0) * QB)
    valid = lane < KCH
    ids_ref[...] = jnp.where(valid, srt, 0)
    fidx_ref[...] = jnp.where(valid, srt + qrow * NCH, 0)


def _select_chunks(cm2d):
    return pl.pallas_call(
        _select_body,
        grid=(Q // QB,),
        in_specs=[pl.BlockSpec((QB, NCHP), lambda q: (q, 0))],
        out_specs=[
            pl.BlockSpec((QB, 128), lambda q: (q, 0)),
            pl.BlockSpec((QB, 128), lambda q: (q, 0)),
        ],
        out_shape=[
            jax.ShapeDtypeStruct((Q, 128), jnp.int32),
            jax.ShapeDtypeStruct((Q, 128), jnp.int32),
        ],
    )(cm2d)


_NC = 2
_NS = 16
_NW = _NC * _NS
_ROWS = Q * KCH
_RPW = _ROWS // _NW
_SUB = 128


def _gather_body(d2_tab, y_tab, idx_d2, idx_lab,
                 out_d2, out_lab, idxf, idxl, rows_f, rows_i,
                 gf, gi, sf, si):
    wid = lax.axis_index("s") * _NC + lax.axis_index("c")
    nb = _RPW // _SUB
    for b in range(nb):
        base = wid * _RPW + b * _SUB
        pltpu.sync_copy(idx_d2.at[pl.ds(base, _SUB)], idxf.at[b])
        pltpu.sync_copy(idx_lab.at[pl.ds(base, _SUB)], idxl.at[b])
    for b in range(nb):
        base = wid * _RPW + b * _SUB
        hf = pltpu.async_copy(d2_tab.at[idxf.at[b]], rows_f, gf)
        hi = pltpu.async_copy(y_tab.at[idxl.at[b]], rows_i, gi)
        hf.wait()
        hsf = pltpu.async_copy(rows_f, out_d2.at[pl.ds(base, _SUB)], sf)
        hi.wait()
        hsi = pltpu.async_copy(rows_i, out_lab.at[pl.ds(base, _SUB)], si)
        hsf.wait()
        hsi.wait()


def _gather_candidates(d2_tab, y_tab, idx_d2, idx_lab):
    mesh = plsc.VectorSubcoreMesh(core_axis_name="c", subcore_axis_name="s")
    f = functools.partial(
        pl.kernel,
        mesh=mesh,
        out_type=[
            jax.ShapeDtypeStruct((_ROWS, CH), jnp.float32),
            jax.ShapeDtypeStruct((_ROWS, CH), jnp.int32),
        ],
        scratch_types=[
            pltpu.VMEM((_RPW // _SUB, _SUB), jnp.int32),
            pltpu.VMEM((_RPW // _SUB, _SUB), jnp.int32),
            pltpu.VMEM((_SUB, CH), jnp.float32),
            pltpu.VMEM((_SUB, CH), jnp.int32),
            pltpu.SemaphoreType.DMA,
            pltpu.SemaphoreType.DMA,
            pltpu.SemaphoreType.DMA,
            pltpu.SemaphoreType.DMA,
        ],
    )(_gather_body)
    return f(d2_tab, y_tab, idx_d2, idx_lab)


def _vote_body(d2c_ref, lab_ref, preds_ref, probs_ref):
    v = d2c_ref[...]
    labs = lab_ref[...]
    lane = lax.broadcasted_iota(jnp.int32, (QB, CAND), 1)
    cls = lax.broadcasted_iota(jnp.int32, (QB, 128), 1)
    votes = jnp.zeros((QB, 128), jnp.float32)
    for _ in range(NN):
        m = jnp.min(v, axis=1, keepdims=True)
        pos = jnp.min(jnp.where(v == m, lane, _BIG_I), axis=1, keepdims=True)
        sel = lane == pos
        labsel = jnp.sum(jnp.where(sel, labs, 0), axis=1, keepdims=True)
        votes = votes + jnp.where(cls == labsel, 1.0, 0.0)
        v = jnp.where(sel, _BIG_F, v)
    probs_ref[...] = votes / float(NN)
    pv = jnp.where(cls < NCLS, votes, -1.0)
    mx = jnp.max(pv, axis=1, keepdims=True)
    pred = jnp.min(jnp.where(pv == mx, cls, _BIG_I), axis=1, keepdims=True)
    preds_ref[...] = jnp.broadcast_to(pred, (QB, 128))


def _vote(cand_d2, cand_lab):
    return pl.pallas_call(
        _vote_body,
        grid=(Q // QB,),
        in_specs=[
            pl.BlockSpec((QB, CAND), lambda q: (q, 0)),
            pl.BlockSpec((QB, CAND), lambda q: (q, 0)),
        ],
        out_specs=[
            pl.BlockSpec((QB, 128), lambda q: (q, 0)),
            pl.BlockSpec((QB, 128), lambda q: (q, 0)),
        ],
        out_shape=[
            jax.ShapeDtypeStruct((Q, 128), jnp.int32),
            jax.ShapeDtypeStruct((Q, 128), jnp.float32),
        ],
    )(cand_d2, cand_lab)


def kernel(x, train_x, train_y):
    x_sq = jnp.sum(x * x, axis=1, keepdims=True)
    t_sq = jnp.sum(train_x * train_x, axis=1)
    xsq_t = jnp.broadcast_to(x_sq, (Q, 128))
    t_sq_pad = jnp.concatenate(
        [t_sq, jnp.full((NPAD - N,), 1e9, jnp.float32)]).reshape(NJ, 1, TB)
    ty_pad = jnp.concatenate(
        [train_y, jnp.zeros((NPAD - N,), jnp.int32)]).reshape(NCH, CH)

    d2, cm3 = _distances(x, train_x, xsq_t, t_sq_pad)
    cm2d = cm3.transpose(1, 0, 2)[:, :, :TB // CH].reshape(Q, NCH)
    cm2d = jnp.concatenate(
        [cm2d, jnp.full((Q, NCHP - NCH), 2e30, jnp.float32)], axis=1)

    ids_pad, fidx_pad = _select_chunks(cm2d)
    idx_d2 = fidx_pad[:, :KCH].reshape(_ROWS)
    idx_lab = ids_pad[:, :KCH].reshape(_ROWS)

    cand_d2, cand_lab = _gather_candidates(
        d2.reshape(Q * NCH, CH), ty_pad, idx_d2, idx_lab)

    preds_pad, probs_pad = _vote(
        cand_d2.reshape(Q, CAND), cand_lab.reshape(Q, CAND))
    return preds_pad[:, 0], probs_pad[:, :NCLS]


# speedup vs baseline: 5.3897x; 1.0141x over previous
import functools

import jax
import jax.numpy as jnp
from jax import lax
from jax.experimental import pallas as pl
from jax.experimental.pallas import tpu as pltpu
from jax.experimental.pallas import tpu_sc as plsc

Q = 1024
D = 128
N = 100000
CH = 128
TB = 2048
QB = 256
NPAD = 100352
NJ = NPAD // TB
NCH = NPAD // CH
NCHP = 896
KCH = 16
CAND = KCH * CH
NN = 15
NCLS = 10

_BIG_F = 3.0e38
_BIG_I = 1 << 30


def _dist_body(x_ref, tx_ref, xsq_ref, tsq_ref, d2_ref, cm_ref):
    xb = x_ref[...]
    tb = tx_ref[...]
    mm = lax.dot_general(xb, tb, (((1,), (1,)), ((), ())),
                         preferred_element_type=jnp.float32)
    tsq = tsq_ref[0, 0, :][None, :]
    xsq = xsq_ref[:, 0:1]
    d2 = (xsq + tsq) - 2.0 * mm
    gcol = lax.broadcasted_iota(jnp.int32, (QB, TB), 1) + pl.program_id(0) * TB
    d2 = jnp.where(gcol < N, d2, 1e9)
    d2_ref[...] = d2
    lane = lax.broadcasted_iota(jnp.int32, (QB, 128), 1)
    cm = jnp.zeros((QB, 128), jnp.float32)
    for c in range(TB // CH):
        mins = jnp.min(d2[:, c * CH:(c + 1) * CH], axis=1)
        cm = cm + jnp.where(lane == c, mins[:, None], 0.0)
    cm_ref[...] = cm.reshape(1, QB, 128)


def _distances(x, tx_pad, xsq_t, tsq_r):
    return pl.pallas_call(
        _dist_body,
        grid=(NJ, Q // QB),
        in_specs=[
            pl.BlockSpec((QB, D), lambda j, q: (q, 0)),
            pl.BlockSpec((TB, D), lambda j, q: (j, 0)),
            pl.BlockSpec((QB, 128), lambda j, q: (q, 0)),
            pl.BlockSpec((1, 1, TB), lambda j, q: (j, 0, 0)),
        ],
        out_specs=[
            pl.BlockSpec((QB, TB), lambda j, q: (q, j)),
            pl.BlockSpec((1, QB, 128), lambda j, q: (j, q, 0)),
        ],
        out_shape=[
            jax.ShapeDtypeStruct((Q, NPAD), jnp.float32),
            jax.ShapeDtypeStruct((NJ, Q, 128), jnp.float32),
        ],
    )(x, tx_pad, xsq_t, tsq_r)


def _select_body(cm_ref, ids_ref, fidx_ref):
    w = cm_ref[...]
    col = lax.broadcasted_iota(jnp.int32, (QB, NCHP), 1)
    lane = lax.broadcasted_iota(jnp.int32, (QB, 128), 1)
    ids = jnp.zeros((QB, 128), jnp.int32)
    for i in range(KCH):
        m = jnp.min(w, axis=1, keepdims=True)
        first = jnp.min(jnp.where(w == m, col, _BIG_I), axis=1, keepdims=True)
        ids = ids + jnp.where(lane == i, first, 0)
        w = jnp.where(col == first, _BIG_F, w)
    s = jnp.where(lane < KCH, ids, _BIG_I)
    srt = jnp.zeros((QB, 128), jnp.int32)
    for j in range(KCH):
        mn = jnp.min(s, axis=1, keepdims=True)
        srt = srt + jnp.where(lane == j, mn, 0)
        s = jnp.where(s == mn, _BIG_I, s)
    qrow = (lax.broadcasted_iota(jnp.int32, (QB, 128), 0)
            + pl.program_id(0) * QB)
    valid = lane < KCH
    ids_ref[...] = jnp.where(valid, srt, 0)
    fidx_ref[...] = jnp.where(valid, srt + qrow * NCH, 0)


def _select_chunks(cm2d):
    return pl.pallas_call(
        _select_body,
        grid=(Q // QB,),
        in_specs=[pl.BlockSpec((QB, NCHP), lambda q: (q, 0))],
        out_specs=[
            pl.BlockSpec((QB, 128), lambda q: (q, 0)),
            pl.BlockSpec((QB, 128), lambda q: (q, 0)),
        ],
        out_shape=[
            jax.ShapeDtypeStruct((Q, 128), jnp.int32),
            jax.ShapeDtypeStruct((Q, 128), jnp.int32),
        ],
    )(cm2d)


_NC = 2
_NS = 16
_NW = _NC * _NS
_ROWS = Q * KCH
_RPW = _ROWS // _NW
_SUB = 128


def _gather_body(d2_tab, y_tab, idx_d2, idx_lab,
                 out_d2, out_lab, idxf, idxl, rows_f, rows_i,
                 gf, gi, sf, si):
    wid = lax.axis_index("s") * _NC + lax.axis_index("c")
    nb = _RPW // _SUB
    for b in range(nb):
        base = wid * _RPW + b * _SUB
        pltpu.sync_copy(idx_d2.at[pl.ds(base, _SUB)], idxf.at[b])
        pltpu.sync_copy(idx_lab.at[pl.ds(base, _SUB)], idxl.at[b])
    for b in range(1):
        base = wid * _RPW + b * _SUB
        hf = pltpu.async_copy(d2_tab.at[idxf.at[b]], rows_f, gf)
        hi = pltpu.async_copy(y_tab.at[idxl.at[b]], rows_i, gi)
        hf.wait()
        hsf = pltpu.async_copy(rows_f, out_d2.at[pl.ds(base, _SUB)], sf)
        hi.wait()
        hsi = pltpu.async_copy(rows_i, out_lab.at[pl.ds(base, _SUB)], si)
        hsf.wait()
        hsi.wait()


def _gather_candidates(d2_tab, y_tab, idx_d2, idx_lab):
    mesh = plsc.VectorSubcoreMesh(core_axis_name="c", subcore_axis_name="s")
    f = functools.partial(
        pl.kernel,
        mesh=mesh,
        out_type=[
            jax.ShapeDtypeStruct((_ROWS, CH), jnp.float32),
            jax.ShapeDtypeStruct((_ROWS, CH), jnp.int32),
        ],
        scratch_types=[
            pltpu.VMEM((_RPW // _SUB, _SUB), jnp.int32),
            pltpu.VMEM((_RPW // _SUB, _SUB), jnp.int32),
            pltpu.VMEM((_SUB, CH), jnp.float32),
            pltpu.VMEM((_SUB, CH), jnp.int32),
            pltpu.SemaphoreType.DMA,
            pltpu.SemaphoreType.DMA,
            pltpu.SemaphoreType.DMA,
            pltpu.SemaphoreType.DMA,
        ],
    )(_gather_body)
    return f(d2_tab, y_tab, idx_d2, idx_lab)


def _vote_body(d2c_ref, lab_ref, preds_ref, probs_ref):
    v = d2c_ref[...]
    labs = lab_ref[...]
    lane = lax.broadcasted_iota(jnp.int32, (QB, CAND), 1)
    cls = lax.broadcasted_iota(jnp.int32, (QB, 128), 1)
    votes = jnp.zeros((QB, 128), jnp.float32)
    for _ in range(NN):
        m = jnp.min(v, axis=1, keepdims=True)
        pos = jnp.min(jnp.where(v == m, lane, _BIG_I), axis=1, keepdims=True)
        sel = lane == pos
        labsel = jnp.sum(jnp.where(sel, labs, 0), axis=1, keepdims=True)
        votes = votes + jnp.where(cls == labsel, 1.0, 0.0)
        v = jnp.where(sel, _BIG_F, v)
    probs_ref[...] = votes / float(NN)
    pv = jnp.where(cls < NCLS, votes, -1.0)
    mx = jnp.max(pv, axis=1, keepdims=True)
    pred = jnp.min(jnp.where(pv == mx, cls, _BIG_I), axis=1, keepdims=True)
    preds_ref[...] = jnp.broadcast_to(pred, (QB, 128))


def _vote(cand_d2, cand_lab):
    return pl.pallas_call(
        _vote_body,
        grid=(Q // QB,),
        in_specs=[
            pl.BlockSpec((QB, CAND), lambda q: (q, 0)),
            pl.BlockSpec((QB, CAND), lambda q: (q, 0)),
        ],
        out_specs=[
            pl.BlockSpec((QB, 128), lambda q: (q, 0)),
            pl.BlockSpec((QB, 128), lambda q: (q, 0)),
        ],
        out_shape=[
            jax.ShapeDtypeStruct((Q, 128), jnp.int32),
            jax.ShapeDtypeStruct((Q, 128), jnp.float32),
        ],
    )(cand_d2, cand_lab)


def kernel(x, train_x, train_y):
    x_sq = jnp.sum(x * x, axis=1, keepdims=True)
    t_sq = jnp.sum(train_x * train_x, axis=1)
    xsq_t = jnp.broadcast_to(x_sq, (Q, 128))
    t_sq_pad = jnp.concatenate(
        [t_sq, jnp.full((NPAD - N,), 1e9, jnp.float32)]).reshape(NJ, 1, TB)
    ty_pad = jnp.concatenate(
        [train_y, jnp.zeros((NPAD - N,), jnp.int32)]).reshape(NCH, CH)

    d2, cm3 = _distances(x, train_x, xsq_t, t_sq_pad)
    cm2d = cm3.transpose(1, 0, 2)[:, :, :TB // CH].reshape(Q, NCH)
    cm2d = jnp.concatenate(
        [cm2d, jnp.full((Q, NCHP - NCH), 2e30, jnp.float32)], axis=1)

    ids_pad, fidx_pad = _select_chunks(cm2d)
    idx_d2 = fidx_pad[:, :KCH].reshape(_ROWS)
    idx_lab = ids_pad[:, :KCH].reshape(_ROWS)

    cand_d2, cand_lab = _gather_candidates(
        d2.reshape(Q * NCH, CH), ty_pad, idx_d2, idx_lab)

    preds_pad, probs_pad = _vote(
        cand_d2.reshape(Q, CAND), cand_lab.reshape(Q, CAND))
    return preds_pad[:, 0], probs_pad[:, :NCLS]


# speedup vs baseline: 6.4788x; 1.2021x over previous
import functools

import jax
import jax.numpy as jnp
from jax import lax
from jax.experimental import pallas as pl
from jax.experimental.pallas import tpu as pltpu
from jax.experimental.pallas import tpu_sc as plsc

Q = 1024
D = 128
N = 100000
CH = 128
TB = 2048
QB = 256
NPAD = 100352
NJ = NPAD // TB
NCH = NPAD // CH
NCHP = 896
KCH = 16
CAND = KCH * CH
NN = 15
NCLS = 10

_BIG_F = 3.0e38
_BIG_I = 1 << 30


def _dist_body(x_ref, tx_ref, xsq_ref, tsq_ref, d2_ref, cm_ref):
    xb = x_ref[...]
    tb = tx_ref[...]
    mm = lax.dot_general(xb, tb, (((1,), (1,)), ((), ())),
                         preferred_element_type=jnp.float32)
    tsq = tsq_ref[0, 0, :][None, :]
    xsq = xsq_ref[:, 0:1]
    d2 = (xsq + tsq) - 2.0 * mm
    gcol = lax.broadcasted_iota(jnp.int32, (Q, TB), 1) + pl.program_id(0) * TB
    d2 = jnp.where(gcol < N, d2, 1e9)
    d2_ref[...] = d2
    lane = lax.broadcasted_iota(jnp.int32, (Q, 128), 1)
    cm = jnp.zeros((Q, 128), jnp.float32)
    for c in range(TB // CH):
        mins = jnp.min(d2[:, c * CH:(c + 1) * CH], axis=1)
        cm = cm + jnp.where(lane == c, mins[:, None], 0.0)
    cm_ref[...] = cm.reshape(1, Q, 128)


def _distances(x, tx_pad, xsq_t, tsq_r):
    return pl.pallas_call(
        _dist_body,
        grid=(NJ,),
        in_specs=[
            pl.BlockSpec((Q, D), lambda j: (0, 0)),
            pl.BlockSpec((TB, D), lambda j: (j, 0)),
            pl.BlockSpec((Q, 128), lambda j: (0, 0)),
            pl.BlockSpec((1, 1, TB), lambda j: (j, 0, 0)),
        ],
        out_specs=[
            pl.BlockSpec((Q, TB), lambda j: (0, j)),
            pl.BlockSpec((1, Q, 128), lambda j: (j, 0, 0)),
        ],
        out_shape=[
            jax.ShapeDtypeStruct((Q, NPAD), jnp.float32),
            jax.ShapeDtypeStruct((NJ, Q, 128), jnp.float32),
        ],
    )(x, tx_pad, xsq_t, tsq_r)


def _select_body(cm_ref, ids_ref, fidx_ref):
    w = cm_ref[...]
    col = lax.broadcasted_iota(jnp.int32, (QB, NCH), 1)
    lane = lax.broadcasted_iota(jnp.int32, (QB, 128), 1)
    ids = jnp.zeros((QB, 128), jnp.int32)
    for i in range(KCH):
        m = jnp.min(w, axis=1, keepdims=True)
        first = jnp.min(jnp.where(w == m, col, _BIG_I), axis=1, keepdims=True)
        ids = ids + jnp.where(lane == i, first, 0)
        w = jnp.where(col == first, _BIG_F, w)
    s = jnp.where(lane < KCH, ids, _BIG_I)
    srt = jnp.zeros((QB, 128), jnp.int32)
    for j in range(KCH):
        mn = jnp.min(s, axis=1, keepdims=True)
        srt = srt + jnp.where(lane == j, mn, 0)
        s = jnp.where(s == mn, _BIG_I, s)
    qrow = (lax.broadcasted_iota(jnp.int32, (QB, 128), 0)
            + pl.program_id(0) * QB)
    valid = lane < KCH
    ids_ref[...] = jnp.where(valid, srt, 0)
    fidx_ref[...] = jnp.where(valid, srt + qrow * NCH, 0)


def _select_chunks(cm2d):
    return pl.pallas_call(
        _select_body,
        grid=(Q // QB,),
        in_specs=[pl.BlockSpec((QB, NCH), lambda q: (q, 0))],
        out_specs=[
            pl.BlockSpec((QB, 128), lambda q: (q, 0)),
            pl.BlockSpec((QB, 128), lambda q: (q, 0)),
        ],
        out_shape=[
            jax.ShapeDtypeStruct((Q, 128), jnp.int32),
            jax.ShapeDtypeStruct((Q, 128), jnp.int32),
        ],
    )(cm2d)


_NC = 2
_NS = 16
_NW = _NC * _NS
_ROWS = Q * KCH
_RPW = _ROWS // _NW
_SUB = 128


def _gather_body(d2_tab, y_tab, idx_d2, idx_lab,
                 out_d2, out_lab, idxf, idxl, rows_f, rows_i,
                 gf, gi, sf, si):
    wid = lax.axis_index("s") * _NC + lax.axis_index("c")
    nb = _RPW // _SUB
    for b in range(nb):
        base = wid * _RPW + b * _SUB
        pltpu.sync_copy(idx_d2.at[pl.ds(base, _SUB)], idxf.at[b])
        pltpu.sync_copy(idx_lab.at[pl.ds(base, _SUB)], idxl.at[b])
    for b in range(nb):
        base = wid * _RPW + b * _SUB
        hf = pltpu.async_copy(d2_tab.at[idxf.at[b]], rows_f, gf)
        hi = pltpu.async_copy(y_tab.at[idxl.at[b]], rows_i, gi)
        hf.wait()
        hsf = pltpu.async_copy(rows_f, out_d2.at[pl.ds(base, _SUB)], sf)
        hi.wait()
        hsi = pltpu.async_copy(rows_i, out_lab.at[pl.ds(base, _SUB)], si)
        hsf.wait()
        hsi.wait()


def _gather_candidates(d2_tab, y_tab, idx_d2, idx_lab):
    mesh = plsc.VectorSubcoreMesh(core_axis_name="c", subcore_axis_name="s")
    f = functools.partial(
        pl.kernel,
        mesh=mesh,
        out_type=[
            jax.ShapeDtypeStruct((_ROWS, CH), jnp.float32),
            jax.ShapeDtypeStruct((_ROWS, CH), jnp.int32),
        ],
        scratch_types=[
            pltpu.VMEM((_RPW // _SUB, _SUB), jnp.int32),
            pltpu.VMEM((_RPW // _SUB, _SUB), jnp.int32),
            pltpu.VMEM((_SUB, CH), jnp.float32),
            pltpu.VMEM((_SUB, CH), jnp.int32),
            pltpu.SemaphoreType.DMA,
            pltpu.SemaphoreType.DMA,
            pltpu.SemaphoreType.DMA,
            pltpu.SemaphoreType.DMA,
        ],
    )(_gather_body)
    return f(d2_tab, y_tab, idx_d2, idx_lab)


def _vote_body(d2c_ref, lab_ref, preds_ref, probs_ref):
    v = d2c_ref[...]
    labs = lab_ref[...]
    lane = lax.broadcasted_iota(jnp.int32, (QB, CAND), 1)
    cls = lax.broadcasted_iota(jnp.int32, (QB, 128), 1)
    votes = jnp.zeros((QB, 128), jnp.float32)
    for _ in range(NN):
        m = jnp.min(v, axis=1, keepdims=True)
        pos = jnp.min(jnp.where(v == m, lane, _BIG_I), axis=1, keepdims=True)
        sel = lane == pos
        labsel = jnp.sum(jnp.where(sel, labs, 0), axis=1, keepdims=True)
        votes = votes + jnp.where(cls == labsel, 1.0, 0.0)
        v = jnp.where(sel, _BIG_F, v)
    probs_ref[...] = votes / float(NN)
    pv = jnp.where(cls < NCLS, votes, -1.0)
    mx = jnp.max(pv, axis=1, keepdims=True)
    pred = jnp.min(jnp.where(pv == mx, cls, _BIG_I), axis=1, keepdims=True)
    preds_ref[...] = jnp.broadcast_to(pred, (QB, 128))


def _vote(cand_d2, cand_lab):
    return pl.pallas_call(
        _vote_body,
        grid=(Q // QB,),
        in_specs=[
            pl.BlockSpec((QB, CAND), lambda q: (q, 0)),
            pl.BlockSpec((QB, CAND), lambda q: (q, 0)),
        ],
        out_specs=[
            pl.BlockSpec((QB, 128), lambda q: (q, 0)),
            pl.BlockSpec((QB, 128), lambda q: (q, 0)),
        ],
        out_shape=[
            jax.ShapeDtypeStruct((Q, 128), jnp.int32),
            jax.ShapeDtypeStruct((Q, 128), jnp.float32),
        ],
    )(cand_d2, cand_lab)


def kernel(x, train_x, train_y):
    x_sq = jnp.sum(x * x, axis=1, keepdims=True)
    t_sq = jnp.sum(train_x * train_x, axis=1)
    xsq_t = jnp.broadcast_to(x_sq, (Q, 128))
    t_sq_pad = jnp.concatenate(
        [t_sq, jnp.full((NPAD - N,), 1e9, jnp.float32)]).reshape(NJ, 1, TB)
    ty_pad = jnp.concatenate(
        [train_y, jnp.zeros((NPAD - N,), jnp.int32)]).reshape(NCH, CH)

    d2, cm3 = _distances(x, train_x, xsq_t, t_sq_pad)
    cm2d = cm3.transpose(1, 0, 2)[:, :, :TB // CH].reshape(Q, NCH)

    ids_pad, fidx_pad = _select_chunks(cm2d)
    idx_d2 = fidx_pad[:, :KCH].reshape(_ROWS)
    idx_lab = ids_pad[:, :KCH].reshape(_ROWS)

    cand_d2, cand_lab = _gather_candidates(
        d2.reshape(Q * NCH, CH), ty_pad, idx_d2, idx_lab)

    preds_pad, probs_pad = _vote(
        cand_d2.reshape(Q, CAND), cand_lab.reshape(Q, CAND))
    return preds_pad[:, 0], probs_pad[:, :NCLS]


# speedup vs baseline: 6.4995x; 1.0032x over previous
import functools

import jax
import jax.numpy as jnp
from jax import lax
from jax.experimental import pallas as pl
from jax.experimental.pallas import tpu as pltpu
from jax.experimental.pallas import tpu_sc as plsc

Q = 1024
D = 128
N = 100000
CH = 128
TB = 3584
QB = 256
NPAD = 100352
NJ = NPAD // TB
NCH = NPAD // CH
NCHP = 896
KCH = 16
CAND = KCH * CH
NN = 15
NCLS = 10

_BIG_F = 3.0e38
_BIG_I = 1 << 30


def _dist_body(x_ref, tx_ref, xsq_ref, tsq_ref, d2_ref, cm_ref):
    xb = x_ref[...]
    tb = tx_ref[...]
    mm = lax.dot_general(xb, tb, (((1,), (1,)), ((), ())),
                         preferred_element_type=jnp.float32)
    tsq = tsq_ref[0, 0, :][None, :]
    xsq = xsq_ref[:, 0:1]
    d2 = (xsq + tsq) - 2.0 * mm
    gcol = lax.broadcasted_iota(jnp.int32, (Q, TB), 1) + pl.program_id(0) * TB
    d2 = jnp.where(gcol < N, d2, 1e9)
    d2_ref[...] = d2
    lane = lax.broadcasted_iota(jnp.int32, (Q, 128), 1)
    cm = jnp.zeros((Q, 128), jnp.float32)
    for c in range(TB // CH):
        mins = jnp.min(d2[:, c * CH:(c + 1) * CH], axis=1)
        cm = cm + jnp.where(lane == c, mins[:, None], 0.0)
    cm_ref[...] = cm


def _distances(x, tx_pad, xsq_t, tsq_r):
    return pl.pallas_call(
        _dist_body,
        grid=(NJ,),
        in_specs=[
            pl.BlockSpec((Q, D), lambda j: (0, 0)),
            pl.BlockSpec((TB, D), lambda j: (j, 0)),
            pl.BlockSpec((Q, 128), lambda j: (0, 0)),
            pl.BlockSpec((1, 1, TB), lambda j: (j, 0, 0)),
        ],
        out_specs=[
            pl.BlockSpec((Q, TB), lambda j: (0, j)),
            pl.BlockSpec((Q, 128), lambda j: (0, j)),
        ],
        out_shape=[
            jax.ShapeDtypeStruct((Q, NPAD), jnp.float32),
            jax.ShapeDtypeStruct((Q, NJ * 128), jnp.float32),
        ],
    )(x, tx_pad, xsq_t, tsq_r)


def _select_body(cm_ref, ids_ref, fidx_ref):
    w = cm_ref[...]
    col = lax.broadcasted_iota(jnp.int32, (QB, NCH), 1)
    lane = lax.broadcasted_iota(jnp.int32, (QB, 128), 1)
    ids = jnp.zeros((QB, 128), jnp.int32)
    for i in range(KCH):
        m = jnp.min(w, axis=1, keepdims=True)
        first = jnp.min(jnp.where(w == m, col, _BIG_I), axis=1, keepdims=True)
        ids = ids + jnp.where(lane == i, first, 0)
        w = jnp.where(col == first, _BIG_F, w)
    s = jnp.where(lane < KCH, ids, _BIG_I)
    srt = jnp.zeros((QB, 128), jnp.int32)
    for j in range(KCH):
        mn = jnp.min(s, axis=1, keepdims=True)
        srt = srt + jnp.where(lane == j, mn, 0)
        s = jnp.where(s == mn, _BIG_I, s)
    qrow = (lax.broadcasted_iota(jnp.int32, (QB, 128), 0)
            + pl.program_id(0) * QB)
    valid = lane < KCH
    ids_ref[...] = jnp.where(valid, srt, 0)
    fidx_ref[...] = jnp.where(valid, srt + qrow * NCH, 0)


def _select_chunks(cm2d):
    return pl.pallas_call(
        _select_body,
        grid=(Q // QB,),
        in_specs=[pl.BlockSpec((QB, NCH), lambda q: (q, 0))],
        out_specs=[
            pl.BlockSpec((QB, 128), lambda q: (q, 0)),
            pl.BlockSpec((QB, 128), lambda q: (q, 0)),
        ],
        out_shape=[
            jax.ShapeDtypeStruct((Q, 128), jnp.int32),
            jax.ShapeDtypeStruct((Q, 128), jnp.int32),
        ],
    )(cm2d)


_NC = 2
_NS = 16
_NW = _NC * _NS
_ROWS = Q * KCH
_RPW = _ROWS // _NW
_SUB = 128


def _gather_body(d2_tab, y_tab, idx_d2, idx_lab,
                 out_d2, out_lab, idxf, idxl, rows_f, rows_i,
                 gf, gi, sf, si):
    wid = lax.axis_index("s") * _NC + lax.axis_index("c")
    nb = _RPW // _SUB
    for b in range(nb):
        base = wid * _RPW + b * _SUB
        pltpu.sync_copy(idx_d2.at[pl.ds(base, _SUB)], idxf.at[b])
        pltpu.sync_copy(idx_lab.at[pl.ds(base, _SUB)], idxl.at[b])
    for b in range(nb):
        base = wid * _RPW + b * _SUB
        hf = pltpu.async_copy(d2_tab.at[idxf.at[b]], rows_f, gf)
        hi = pltpu.async_copy(y_tab.at[idxl.at[b]], rows_i, gi)
        hf.wait()
        hsf = pltpu.async_copy(rows_f, out_d2.at[pl.ds(base, _SUB)], sf)
        hi.wait()
        hsi = pltpu.async_copy(rows_i, out_lab.at[pl.ds(base, _SUB)], si)
        hsf.wait()
        hsi.wait()


def _gather_candidates(d2_tab, y_tab, idx_d2, idx_lab):
    mesh = plsc.VectorSubcoreMesh(core_axis_name="c", subcore_axis_name="s")
    f = functools.partial(
        pl.kernel,
        mesh=mesh,
        out_type=[
            jax.ShapeDtypeStruct((_ROWS, CH), jnp.float32),
            jax.ShapeDtypeStruct((_ROWS, CH), jnp.int32),
        ],
        scratch_types=[
            pltpu.VMEM((_RPW // _SUB, _SUB), jnp.int32),
            pltpu.VMEM((_RPW // _SUB, _SUB), jnp.int32),
            pltpu.VMEM((_SUB, CH), jnp.float32),
            pltpu.VMEM((_SUB, CH), jnp.int32),
            pltpu.SemaphoreType.DMA,
            pltpu.SemaphoreType.DMA,
            pltpu.SemaphoreType.DMA,
            pltpu.SemaphoreType.DMA,
        ],
    )(_gather_body)
    return f(d2_tab, y_tab, idx_d2, idx_lab)


def _vote_body(d2c_ref, lab_ref, preds_ref, probs_ref):
    v = d2c_ref[...]
    labs = lab_ref[...]
    lane = lax.broadcasted_iota(jnp.int32, (QB, CAND), 1)
    cls = lax.broadcasted_iota(jnp.int32, (QB, 128), 1)
    votes = jnp.zeros((QB, 128), jnp.float32)
    for _ in range(NN):
        m = jnp.min(v, axis=1, keepdims=True)
        pos = jnp.min(jnp.where(v == m, lane, _BIG_I), axis=1, keepdims=True)
        sel = lane == pos
        labsel = jnp.sum(jnp.where(sel, labs, 0), axis=1, keepdims=True)
        votes = votes + jnp.where(cls == labsel, 1.0, 0.0)
        v = jnp.where(sel, _BIG_F, v)
    probs_ref[...] = votes / float(NN)
    pv = jnp.where(cls < NCLS, votes, -1.0)
    mx = jnp.max(pv, axis=1, keepdims=True)
    pred = jnp.min(jnp.where(pv == mx, cls, _BIG_I), axis=1, keepdims=True)
    preds_ref[...] = jnp.broadcast_to(pred, (QB, 128))


def _vote(cand_d2, cand_lab):
    return pl.pallas_call(
        _vote_body,
        grid=(Q // QB,),
        in_specs=[
            pl.BlockSpec((QB, CAND), lambda q: (q, 0)),
            pl.BlockSpec((QB, CAND), lambda q: (q, 0)),
        ],
        out_specs=[
            pl.BlockSpec((QB, 128), lambda q: (q, 0)),
            pl.BlockSpec((QB, 128), lambda q: (q, 0)),
        ],
        out_shape=[
            jax.ShapeDtypeStruct((Q, 128), jnp.int32),
            jax.ShapeDtypeStruct((Q, 128), jnp.float32),
        ],
    )(cand_d2, cand_lab)


def kernel(x, train_x, train_y):
    x_sq = jnp.sum(x * x, axis=1, keepdims=True)
    t_sq = jnp.sum(train_x * train_x, axis=1)
    xsq_t = jnp.broadcast_to(x_sq, (Q, 128))
    t_sq_pad = jnp.concatenate(
        [t_sq, jnp.full((NPAD - N,), 1e9, jnp.float32)]).reshape(NJ, 1, TB)
    ty_pad = jnp.concatenate(
        [train_y, jnp.zeros((NPAD - N,), jnp.int32)]).reshape(NCH, CH)

    d2, cm3 = _distances(x, train_x, xsq_t, t_sq_pad)
    cm2d = cm3.reshape(Q, NJ, 128)[:, :, :TB // CH].reshape(Q, NCH)

    ids_pad, fidx_pad = _select_chunks(cm2d)
    idx_d2 = fidx_pad[:, :KCH].reshape(_ROWS)
    idx_lab = ids_pad[:, :KCH].reshape(_ROWS)

    cand_d2, cand_lab = _gather_candidates(
        d2.reshape(Q * NCH, CH), ty_pad, idx_d2, idx_lab)

    preds_pad, probs_pad = _vote(
        cand_d2.reshape(Q, CAND), cand_lab.reshape(Q, CAND))
    return preds_pad[:, 0], probs_pad[:, :NCLS]


# speedup vs baseline: 9.7252x; 1.4963x over previous
import functools

import jax
import jax.numpy as jnp
from jax import lax
from jax.experimental import pallas as pl
from jax.experimental.pallas import tpu as pltpu
from jax.experimental.pallas import tpu_sc as plsc

Q = 1024
D = 128
N = 100000
CH = 128
TB = 2048
QB = 256
NPAD = 100352
NJ = NPAD // TB
NCH = NPAD // CH
NCHP = 896
KCH = 16
CAND = KCH * CH
NN = 15
NCLS = 10

_BIG_F = 3.0e38
_BIG_I = 1 << 30


def _dist_body(x_ref, tx_ref, xsq_ref, tsq_ref, d2_ref, cm_ref):
    xb = x_ref[...]
    tb = tx_ref[...]
    mm = lax.dot_general(xb, tb, (((1,), (1,)), ((), ())),
                         preferred_element_type=jnp.float32)
    tsq = tsq_ref[0, 0, :][None, :]
    xsq = xsq_ref[:, 0:1]
    d2 = (xsq + tsq) - 2.0 * mm
    gcol = lax.broadcasted_iota(jnp.int32, (Q, TB), 1) + pl.program_id(0) * TB
    d2 = jnp.where(gcol < N, d2, 1e9)
    d2_ref[...] = d2.reshape(Q, TB // CH, CH)
    lane = lax.broadcasted_iota(jnp.int32, (Q, 128), 1)
    cm = jnp.zeros((Q, 128), jnp.float32)
    for c in range(TB // CH):
        mins = jnp.min(d2[:, c * CH:(c + 1) * CH], axis=1)
        cm = cm + jnp.where(lane == c, mins[:, None], 0.0)
    cm_ref[...] = cm


def _distances(x, tx_pad, xsq_t, tsq_r):
    return pl.pallas_call(
        _dist_body,
        grid=(NJ,),
        in_specs=[
            pl.BlockSpec((Q, D), lambda j: (0, 0)),
            pl.BlockSpec((TB, D), lambda j: (j, 0)),
            pl.BlockSpec((Q, 128), lambda j: (0, 0)),
            pl.BlockSpec((1, 1, TB), lambda j: (j, 0, 0)),
        ],
        out_specs=[
            pl.BlockSpec((Q, TB // CH, CH), lambda j: (0, j, 0)),
            pl.BlockSpec((Q, 128), lambda j: (0, j)),
        ],
        out_shape=[
            jax.ShapeDtypeStruct((Q, NCH, CH), jnp.float32),
            jax.ShapeDtypeStruct((Q, NJ * 128), jnp.float32),
        ],
    )(x, tx_pad, xsq_t, tsq_r)


def _select_body(cm_ref, ids_ref, fidx_ref):
    w = cm_ref[...]
    col = lax.broadcasted_iota(jnp.int32, (QB, NCH), 1)
    lane = lax.broadcasted_iota(jnp.int32, (QB, 128), 1)
    ids = jnp.zeros((QB, 128), jnp.int32)
    for i in range(KCH):
        m = jnp.min(w, axis=1, keepdims=True)
        first = jnp.min(jnp.where(w == m, col, _BIG_I), axis=1, keepdims=True)
        ids = ids + jnp.where(lane == i, first, 0)
        w = jnp.where(col == first, _BIG_F, w)
    s = jnp.where(lane < KCH, ids, _BIG_I)
    srt = jnp.zeros((QB, 128), jnp.int32)
    for j in range(KCH):
        mn = jnp.min(s, axis=1, keepdims=True)
        srt = srt + jnp.where(lane == j, mn, 0)
        s = jnp.where(s == mn, _BIG_I, s)
    qrow = (lax.broadcasted_iota(jnp.int32, (QB, 128), 0)
            + pl.program_id(0) * QB)
    valid = lane < KCH
    ids_ref[...] = jnp.where(valid, srt, 0)
    fidx_ref[...] = jnp.where(valid, srt + qrow * NCH, 0)


def _select_chunks(cm2d):
    return pl.pallas_call(
        _select_body,
        grid=(Q // QB,),
        in_specs=[pl.BlockSpec((QB, NCH), lambda q: (q, 0))],
        out_specs=[
            pl.BlockSpec((QB, 128), lambda q: (q, 0)),
            pl.BlockSpec((QB, 128), lambda q: (q, 0)),
        ],
        out_shape=[
            jax.ShapeDtypeStruct((Q, 128), jnp.int32),
            jax.ShapeDtypeStruct((Q, 128), jnp.int32),
        ],
    )(cm2d)


_NC = 2
_NS = 16
_NW = _NC * _NS
_ROWS = Q * KCH
_RPW = _ROWS // _NW
_SUB = 128


def _gather_body(d2_tab, y_tab, idx_d2, idx_lab,
                 out_d2, out_lab, idxf, idxl, rows_f, rows_i,
                 gf, gi, sf, si):
    wid = lax.axis_index("s") * _NC + lax.axis_index("c")
    nb = _RPW // _SUB
    for b in range(nb):
        base = wid * _RPW + b * _SUB
        pltpu.sync_copy(idx_d2.at[pl.ds(base, _SUB)], idxf.at[b])
        pltpu.sync_copy(idx_lab.at[pl.ds(base, _SUB)], idxl.at[b])
    for b in range(nb):
        base = wid * _RPW + b * _SUB
        hf = pltpu.async_copy(d2_tab.at[idxf.at[b]], rows_f, gf)
        hi = pltpu.async_copy(y_tab.at[idxl.at[b]], rows_i, gi)
        hf.wait()
        hsf = pltpu.async_copy(rows_f, out_d2.at[pl.ds(base, _SUB)], sf)
        hi.wait()
        hsi = pltpu.async_copy(rows_i, out_lab.at[pl.ds(base, _SUB)], si)
        hsf.wait()
        hsi.wait()


def _gather_candidates(d2_tab, y_tab, idx_d2, idx_lab):
    mesh = plsc.VectorSubcoreMesh(core_axis_name="c", subcore_axis_name="s")
    f = functools.partial(
        pl.kernel,
        mesh=mesh,
        out_type=[
            jax.ShapeDtypeStruct((_ROWS, CH), jnp.float32),
            jax.ShapeDtypeStruct((_ROWS, CH), jnp.int32),
        ],
        scratch_types=[
            pltpu.VMEM((_RPW // _SUB, _SUB), jnp.int32),
            pltpu.VMEM((_RPW // _SUB, _SUB), jnp.int32),
            pltpu.VMEM((_SUB, CH), jnp.float32),
            pltpu.VMEM((_SUB, CH), jnp.int32),
            pltpu.SemaphoreType.DMA,
            pltpu.SemaphoreType.DMA,
            pltpu.SemaphoreType.DMA,
            pltpu.SemaphoreType.DMA,
        ],
    )(_gather_body)
    return f(d2_tab, y_tab, idx_d2, idx_lab)


def _vote_body(d2c_ref, lab_ref, preds_ref, probs_ref):
    v = d2c_ref[...]
    labs = lab_ref[...]
    lane = lax.broadcasted_iota(jnp.int32, (QB, CAND), 1)
    cls = lax.broadcasted_iota(jnp.int32, (QB, 128), 1)
    votes = jnp.zeros((QB, 128), jnp.float32)
    for _ in range(NN):
        m = jnp.min(v, axis=1, keepdims=True)
        pos = jnp.min(jnp.where(v == m, lane, _BIG_I), axis=1, keepdims=True)
        sel = lane == pos
        labsel = jnp.sum(jnp.where(sel, labs, 0), axis=1, keepdims=True)
        votes = votes + jnp.where(cls == labsel, 1.0, 0.0)
        v = jnp.where(sel, _BIG_F, v)
    probs_ref[...] = votes / float(NN)
    pv = jnp.where(cls < NCLS, votes, -1.0)
    mx = jnp.max(pv, axis=1, keepdims=True)
    pred = jnp.min(jnp.where(pv == mx, cls, _BIG_I), axis=1, keepdims=True)
    preds_ref[...] = jnp.broadcast_to(pred, (QB, 128))


def _vote(cand_d2, cand_lab):
    return pl.pallas_call(
        _vote_body,
        grid=(Q // QB,),
        in_specs=[
            pl.BlockSpec((QB, CAND), lambda q: (q, 0)),
            pl.BlockSpec((QB, CAND), lambda q: (q, 0)),
        ],
        out_specs=[
            pl.BlockSpec((QB, 128), lambda q: (q, 0)),
            pl.BlockSpec((QB, 128), lambda q: (q, 0)),
        ],
        out_shape=[
            jax.ShapeDtypeStruct((Q, 128), jnp.int32),
            jax.ShapeDtypeStruct((Q, 128), jnp.float32),
        ],
    )(cand_d2, cand_lab)


def kernel(x, train_x, train_y):
    x_sq = jnp.sum(x * x, axis=1, keepdims=True)
    t_sq = jnp.sum(train_x * train_x, axis=1)
    xsq_t = jnp.broadcast_to(x_sq, (Q, 128))
    t_sq_pad = jnp.concatenate(
        [t_sq, jnp.full((NPAD - N,), 1e9, jnp.float32)]).reshape(NJ, 1, TB)
    ty_pad = jnp.concatenate(
        [train_y, jnp.zeros((NPAD - N,), jnp.int32)]).reshape(NCH, CH)

    d2, cm3 = _distances(x, train_x, xsq_t, t_sq_pad)
    cm2d = cm3.reshape(Q, NJ, 128)[:, :, :TB // CH].reshape(Q, NCH)

    ids_pad, fidx_pad = _select_chunks(cm2d)
    idx_d2 = fidx_pad[:, :KCH].reshape(_ROWS)
    idx_lab = ids_pad[:, :KCH].reshape(_ROWS)

    cand_d2, cand_lab = _gather_candidates(
        d2.reshape(Q * NCH, CH), ty_pad, idx_d2, idx_lab)

    preds_pad, probs_pad = _vote(
        cand_d2.reshape(Q, CAND), cand_lab.reshape(Q, CAND))
    return preds_pad[:, 0], probs_pad[:, :NCLS]
